# baseline (speedup 1.0000x reference)
import functools

import jax
import jax.numpy as jnp
from jax.experimental import pallas as pl
from jax.experimental.pallas import tpu as pltpu

_BM = 200


def _gcn_kernel(x_ref, w1_ref, w2_ref, adj_ref, out_ref, s1_ref, s2_ref, *, nb):
    i = pl.program_id(0)

    @pl.when(i == 0)
    def _prologue():
        xb = x_ref[...].astype(jnp.bfloat16)
        w1b = w1_ref[...].astype(jnp.bfloat16)
        s1 = jnp.dot(xb, w1b, preferred_element_type=jnp.float32)
        s1_ref[...] = s1.astype(jnp.bfloat16)

    adj_b = adj_ref[...].astype(jnp.bfloat16)

    @pl.when(i < nb)
    def _layer1():
        h = jnp.dot(adj_b, s1_ref[...], preferred_element_type=jnp.float32)
        h = jnp.maximum(h, 0.0).astype(jnp.bfloat16)
        w2b = w2_ref[...].astype(jnp.bfloat16)
        s2 = jnp.dot(h, w2b, preferred_element_type=jnp.float32)
        s2_ref[pl.ds((i % nb) * _BM, _BM), :] = s2.astype(jnp.bfloat16)

    @pl.when(i >= nb)
    def _layer2():
        out_ref[...] = jnp.dot(adj_b, s2_ref[...],
                               preferred_element_type=jnp.float32)


@jax.jit
def kernel(x, adj, W1, W2):
    n, nfeat = x.shape
    nhid = W1.shape[1]
    nout = W2.shape[1]
    nb = n // _BM

    grid = (2 * nb,)
    return pl.pallas_call(
        functools.partial(_gcn_kernel, nb=nb),
        grid=grid,
        in_specs=[
            pl.BlockSpec((n, nfeat), lambda i: (0, 0)),
            pl.BlockSpec((nfeat, nhid), lambda i: (0, 0)),
            pl.BlockSpec((nhid, nout), lambda i: (0, 0)),
            pl.BlockSpec((_BM, n), lambda i, nb=nb: (i % nb, 0)),
        ],
        out_specs=pl.BlockSpec(
            (_BM, nout),
            lambda i, nb=nb: (jnp.where(i >= nb, i - nb, 0), 0)),
        out_shape=jax.ShapeDtypeStruct((n, nout), jnp.float32),
        scratch_shapes=[
            pltpu.VMEM((n, nhid), jnp.bfloat16),
            pltpu.VMEM((n, nout), jnp.bfloat16),
        ],
        compiler_params=pltpu.CompilerParams(
            vmem_limit_bytes=100 * 1024 * 1024,
        ),
    )(x, W1, W2, adj)


# speedup vs baseline: 1.1104x; 1.1104x over previous
import functools

import jax
import jax.numpy as jnp
from jax.experimental import pallas as pl
from jax.experimental.pallas import tpu as pltpu

_BM = 400
_CK = 2560


def _gcn_kernel(x_ref, w1_ref, w2_ref, adj_ref, out_ref, s1_ref, s2_ref, *, nb):
    i = pl.program_id(0)

    @pl.when(i == 0)
    def _prologue():
        xb = x_ref[...].astype(jnp.bfloat16)
        w1b = w1_ref[...].astype(jnp.bfloat16)
        s1 = jnp.dot(xb, w1b, preferred_element_type=jnp.float32)
        s1_ref[...] = s1.astype(jnp.bfloat16)

    n = adj_ref.shape[1]

    def _adj_dot(rhs_ref):
        acc = None
        for k0 in range(0, n, _CK):
            kw = min(_CK, n - k0)
            a_k = adj_ref[:, k0:k0 + kw].astype(jnp.bfloat16)
            p = jnp.dot(a_k, rhs_ref[k0:k0 + kw, :],
                        preferred_element_type=jnp.float32)
            acc = p if acc is None else acc + p
        return acc

    @pl.when(i < nb)
    def _layer1():
        h = jnp.maximum(_adj_dot(s1_ref), 0.0).astype(jnp.bfloat16)
        w2b = w2_ref[...].astype(jnp.bfloat16)
        s2 = jnp.dot(h, w2b, preferred_element_type=jnp.float32)
        s2_ref[pl.ds((i % nb) * _BM, _BM), :] = s2.astype(jnp.bfloat16)

    @pl.when(i >= nb)
    def _layer2():
        out_ref[...] = _adj_dot(s2_ref)


@jax.jit
def kernel(x, adj, W1, W2):
    n, nfeat = x.shape
    nhid = W1.shape[1]
    nout = W2.shape[1]
    nb = n // _BM

    grid = (2 * nb,)
    return pl.pallas_call(
        functools.partial(_gcn_kernel, nb=nb),
        grid=grid,
        in_specs=[
            pl.BlockSpec((n, nfeat), lambda i: (0, 0)),
            pl.BlockSpec((nfeat, nhid), lambda i: (0, 0)),
            pl.BlockSpec((nhid, nout), lambda i: (0, 0)),
            pl.BlockSpec((_BM, n), lambda i, nb=nb: (i % nb, 0)),
        ],
        out_specs=pl.BlockSpec(
            (_BM, nout),
            lambda i, nb=nb: (jnp.where(i >= nb, i - nb, 0), 0)),
        out_shape=jax.ShapeDtypeStruct((n, nout), jnp.float32),
        scratch_shapes=[
            pltpu.VMEM((n, nhid), jnp.bfloat16),
            pltpu.VMEM((n, nout), jnp.bfloat16),
        ],
        compiler_params=pltpu.CompilerParams(
            vmem_limit_bytes=100 * 1024 * 1024,
        ),
    )(x, W1, W2, adj)


# speedup vs baseline: 1.1127x; 1.0021x over previous
import functools

import jax
import jax.numpy as jnp
from jax.experimental import pallas as pl
from jax.experimental.pallas import tpu as pltpu

_BM = 400


def _gcn_kernel(x_ref, w1_ref, w2_ref, adj_ref, out_ref, s1_ref, s2_ref, *, nb):
    i = pl.program_id(0)

    @pl.when(i == 0)
    def _prologue():
        s1_ref[...] = jnp.dot(x_ref[...], w1_ref[...],
                              preferred_element_type=jnp.float32)

    @pl.when(i < nb)
    def _layer1():
        h = jnp.dot(adj_ref[...], s1_ref[...],
                    preferred_element_type=jnp.float32)
        h = jnp.maximum(h, 0.0)
        s2 = jnp.dot(h, w2_ref[...], preferred_element_type=jnp.float32)
        s2_ref[pl.ds((i % nb) * _BM, _BM), :] = s2

    @pl.when(i >= nb)
    def _layer2():
        out_ref[...] = jnp.dot(adj_ref[...], s2_ref[...],
                               preferred_element_type=jnp.float32)


@jax.jit
def kernel(x, adj, W1, W2):
    n, nfeat = x.shape
    nhid = W1.shape[1]
    nout = W2.shape[1]
    nb = n // _BM

    return pl.pallas_call(
        functools.partial(_gcn_kernel, nb=nb),
        grid=(2 * nb,),
        in_specs=[
            pl.BlockSpec((n, nfeat), lambda i: (0, 0)),
            pl.BlockSpec((nfeat, nhid), lambda i: (0, 0)),
            pl.BlockSpec((nhid, nout), lambda i: (0, 0)),
            pl.BlockSpec((_BM, n), lambda i, nb=nb: (i % nb, 0)),
        ],
        out_specs=pl.BlockSpec(
            (_BM, nout),
            lambda i, nb=nb: (jnp.where(i >= nb, i - nb, 0), 0)),
        out_shape=jax.ShapeDtypeStruct((n, nout), jnp.float32),
        scratch_shapes=[
            pltpu.VMEM((n, nhid), jnp.float32),
            pltpu.VMEM((n, nout), jnp.float32),
        ],
        compiler_params=pltpu.CompilerParams(
            vmem_limit_bytes=100 * 1024 * 1024,
        ),
    )(x, W1, W2, adj)
